# Initial kernel scaffold; baseline (speedup 1.0000x reference)
import jax
import jax.numpy as jnp
from jax.experimental import pallas as pl


def kernel(pred_rets):
    raise NotImplementedError("write your pallas kernel here")


# speedup vs baseline: 10.1923x; 10.1923x over previous
import functools

import jax
import jax.numpy as jnp
from jax.experimental import pallas as pl
from jax.experimental.pallas import tpu as pltpu

_ALPHA = 0.95
_LAMBDA = 1.0
_BLOCK_ROWS = 256
_INT_MIN = -(2 ** 31)


def _cvar_body(nq, x_ref, out_ref, keys_ref):
    i = pl.program_id(0)
    x = x_ref[...]
    rows, cols = x.shape

    bits = jax.lax.bitcast_convert_type(x, jnp.int32)
    keys_ref[...] = bits ^ jnp.bitwise_and(
        jax.lax.shift_right_arithmetic(bits, 31), jnp.int32(0x7FFFFFFF))

    row_sum = jnp.sum(x, axis=1)

    def step(it, prefix):
        bit = jnp.int32(31) - it
        trial = prefix + jnp.left_shift(jnp.int32(1), bit)
        cnt = jnp.sum((keys_ref[...] < trial).astype(jnp.int32), axis=1,
                      keepdims=True)
        return jnp.where(cnt < nq, trial, prefix)

    prefix0 = jnp.full((rows, 1), _INT_MIN, dtype=jnp.int32)
    t_key = jax.lax.fori_loop(0, 32, step, prefix0)

    mask = keys_ref[...] < t_key
    cnt_less = jnp.sum(mask.astype(jnp.float32), axis=1)
    sum_less = jnp.sum(jnp.where(mask, x, 0.0), axis=1)

    t_bits = t_key ^ jnp.bitwise_and(
        jax.lax.shift_right_arithmetic(t_key, 31), jnp.int32(0x7FFFFFFF))
    t_val = jax.lax.bitcast_convert_type(t_bits, jnp.float32)[:, 0]

    tail_sum = sum_less + t_val * (jnp.float32(nq) - cnt_less)
    loss = -row_sum * jnp.float32(1.0 / cols) + \
        _LAMBDA * tail_sum * jnp.float32(1.0 / nq)
    partial = jnp.sum(loss).reshape(1, 1)

    @pl.when(i == 0)
    def _():
        out_ref[...] = jnp.zeros((1, 1), jnp.float32)

    out_ref[...] += partial


def kernel(pred_rets):
    batch, cols = pred_rets.shape
    nq = int(cols * (1 - _ALPHA))
    if nq == 0:
        nq = 1
    block_rows = min(_BLOCK_ROWS, batch)
    grid = batch // block_rows

    out = pl.pallas_call(
        functools.partial(_cvar_body, nq),
        grid=(grid,),
        in_specs=[pl.BlockSpec((block_rows, cols), lambda i: (i, 0))],
        out_specs=pl.BlockSpec((1, 1), lambda i: (0, 0)),
        out_shape=jax.ShapeDtypeStruct((1, 1), jnp.float32),
        scratch_shapes=[pltpu.VMEM((block_rows, cols), jnp.int32)],
    )(pred_rets)
    return jnp.reshape(out, ()) * jnp.float32(1.0 / batch)
